# baseline (speedup 1.0000x reference)
import jax
import jax.numpy as jnp
from jax import lax
from jax.experimental import pallas as pl
from jax.experimental.pallas import tpu as pltpu
from jax.experimental.pallas import tpu_sc as plsc

D = 64
NC, NS = 2, 16
NW = NC * NS
CHUNK = 128
SCALE = 1.0 / (D ** 0.5)


def _make_kernel(n_rows):
    per_w = n_rows // NW
    n_chunks = per_w // CHUNK
    assert n_chunks % 2 == 0
    xrows_per_w = per_w // 128
    orows_per_chunk = CHUNK * D // 128
    mesh = plsc.VectorSubcoreMesh(core_axis_name="c", subcore_axis_name="s")

    def body(x_hbm, table_hbm, out_hbm, idx_v, gidx_v,
             gbuf0, gbuf1, sbuf0, sbuf1, gsem0, gsem1, ssem0, ssem1):
        wid = lax.axis_index("s") * NC + lax.axis_index("c")
        pltpu.sync_copy(x_hbm.at[pl.ds(wid * xrows_per_w, xrows_per_w)], idx_v)

        @plsc.parallel_loop(0, xrows_per_w, unroll=2)
        def _(r):
            for j in range(128 // 16):
                sl = pl.ds(j * 16, 16)
                gidx_v[r, sl] = idx_v[r, sl] >> 1

        gbufs, sbufs = (gbuf0, gbuf1), (sbuf0, sbuf1)
        gsems, ssems = (gsem0, gsem1), (ssem0, ssem1)

        def start_gather(j, b):
            pltpu.async_copy(table_hbm.at[gidx_v.at[j]], gbufs[b], gsems[b])

        def wait_gather(b):
            pltpu.make_async_copy(
                table_hbm.at[gidx_v.at[0]], gbufs[b], gsems[b]).wait()

        def start_scatter(j, b):
            pltpu.async_copy(
                sbufs[b],
                out_hbm.at[pl.ds(
                    pl.multiple_of(
                        wid * (per_w * D // 128) + j * orows_per_chunk, 8),
                    orows_per_chunk)],
                ssems[b])

        def wait_scatter(b):
            pltpu.make_async_copy(
                sbufs[b], out_hbm.at[pl.ds(0, orows_per_chunk)],
                ssems[b]).wait()

        start_gather(0, 0)
        start_gather(1, 1)
        lane = lax.iota(jnp.int32, 16)

        def step(j, b):
            wait_gather(b)

            @pl.when(j >= 2)
            def _():
                wait_scatter(b)

            gbuf, sbuf = gbufs[b], sbufs[b]

            @plsc.parallel_loop(0, CHUNK // 2, unroll=2)
            def _(i2):
                for c16 in range(D // 16):
                    sl = pl.ds(c16 * 16, 16)
                    sh = pl.ds(D + c16 * 16, 16)
                    sbuf[i2, sl] = gbuf[2 * i2, sl] * SCALE
                    sbuf[i2, sh] = gbuf[2 * i2 + 1, sl] * SCALE

            @pl.when(j + 2 < n_chunks)
            def _():
                start_gather(j + 2, b)

            start_scatter(j, b)

        def pair_steps(t, _):
            step(2 * t, 0)
            step(2 * t + 1, 1)
            return 0

        lax.fori_loop(0, n_chunks // 2, pair_steps, 0)
        wait_scatter(0)
        wait_scatter(1)

    return pl.kernel(
        body,
        out_type=jax.ShapeDtypeStruct((n_rows * D // 128, 128), jnp.float32),
        mesh=mesh,
        compiler_params=pltpu.CompilerParams(use_tc_tiling_on_sc=True),
        scratch_types=[
            pltpu.VMEM((xrows_per_w, 128), jnp.int32),
            pltpu.VMEM((xrows_per_w, 128), jnp.int32),
            pltpu.VMEM((CHUNK, 128), jnp.float32),
            pltpu.VMEM((CHUNK, 128), jnp.float32),
            pltpu.VMEM((orows_per_chunk, 128), jnp.float32),
            pltpu.VMEM((orows_per_chunk, 128), jnp.float32),
            pltpu.SemaphoreType.DMA,
            pltpu.SemaphoreType.DMA,
            pltpu.SemaphoreType.DMA,
            pltpu.SemaphoreType.DMA,
        ],
    )


def kernel(x, table):
    B, S = x.shape
    n = B * S
    xr = x.astype(jnp.int32).reshape(n // 128, 128)
    t2 = table.reshape(table.shape[0] // 2, 2 * D)
    out = _make_kernel(n)(xr, t2)
    return out.reshape(B, S, D)


# speedup vs baseline: 1.0704x; 1.0704x over previous
import jax
import jax.numpy as jnp
from jax import lax
from jax.experimental import pallas as pl
from jax.experimental.pallas import tpu as pltpu
from jax.experimental.pallas import tpu_sc as plsc

D = 64
NC, NS = 2, 16
NW = NC * NS
SCALE = 1.0 / (D ** 0.5)
SPLIT = (104, 96)


def _make_kernel(B, S):
    rows_per_w = B // NW
    assert rows_per_w % 2 == 0 and sum(SPLIT) == S
    mesh = plsc.VectorSubcoreMesh(core_axis_name="c", subcore_axis_name="s")

    def body(x_hbm, table_hbm, out_hbm, idx_v,
             gbuf0, gbuf1, sbuf0, sbuf1, gsem0, gsem1, ssem0, ssem1):
        wid = lax.axis_index("s") * NC + lax.axis_index("c")
        row0 = wid * rows_per_w
        pltpu.sync_copy(x_hbm.at[pl.ds(row0, rows_per_w)], idx_v)

        gbufs, sbufs = (gbuf0, gbuf1), (sbuf0, sbuf1)
        gsems, ssems = (gsem0, gsem1), (ssem0, ssem1)

        def gather_descs(r, b):
            descs = []
            s0 = 0
            for w in SPLIT:
                descs.append(pltpu.make_async_copy(
                    table_hbm.at[idx_v.at[r, pl.ds(s0, w)]],
                    gbufs[b].at[pl.ds(s0, w)], gsems[b]))
                s0 += w
            return descs

        def start_gather(r, b):
            for d in gather_descs(r, b):
                d.start()

        def wait_gather(b):
            for d in gather_descs(0, b):
                d.wait()

        def start_scatter(r, b):
            pltpu.async_copy(sbufs[b], out_hbm.at[row0 + r], ssems[b])

        def wait_scatter(b):
            pltpu.make_async_copy(sbufs[b], out_hbm.at[row0], ssems[b]).wait()

        start_gather(0, 0)
        start_gather(1, 1)

        def step(r, b):
            wait_gather(b)

            @pl.when(r >= 2)
            def _():
                wait_scatter(b)

            gbuf, sbuf = gbufs[b], sbufs[b]

            @plsc.parallel_loop(0, S, unroll=4)
            def _(i):
                for j in range(D // 16):
                    sl = pl.ds(j * 16, 16)
                    sbuf[i, sl] = gbuf[i, sl] * SCALE

            @pl.when(r + 2 < rows_per_w)
            def _():
                start_gather(r + 2, b)

            start_scatter(r, b)

        def pair(t, _):
            step(2 * t, 0)
            step(2 * t + 1, 1)
            return 0

        lax.fori_loop(0, rows_per_w // 2, pair, 0)
        wait_scatter(0)
        wait_scatter(1)

    return pl.kernel(
        body,
        out_type=jax.ShapeDtypeStruct((B, S, D), jnp.float32),
        mesh=mesh,
        compiler_params=pltpu.CompilerParams(use_tc_tiling_on_sc=False),
        scratch_types=[
            pltpu.VMEM((rows_per_w, S), jnp.int32),
            pltpu.VMEM((S, D), jnp.float32),
            pltpu.VMEM((S, D), jnp.float32),
            pltpu.VMEM((S, D), jnp.float32),
            pltpu.VMEM((S, D), jnp.float32),
            pltpu.SemaphoreType.DMA,
            pltpu.SemaphoreType.DMA,
            pltpu.SemaphoreType.DMA,
            pltpu.SemaphoreType.DMA,
        ],
    )


def _impl(x, table):
    B, S = x.shape
    return _make_kernel(B, S)(x.astype(jnp.int32), table)


def _jitted():
    from jax.experimental.layout import Format, Layout
    from jax.sharding import SingleDeviceSharding
    fmt = Format(Layout(major_to_minor=(0, 1, 2)),
                 SingleDeviceSharding(jax.devices()[0]))
    return jax.jit(_impl, out_shardings=fmt)


def kernel(x, table):
    return _jitted()(x, table)
